# Initial kernel scaffold; baseline (speedup 1.0000x reference)
import jax
import jax.numpy as jnp
from jax.experimental import pallas as pl


def kernel(j, e, W1, b1, Wc, bc, Wq, bq):
    raise NotImplementedError("write your pallas kernel here")


# speedup vs baseline: 1.9612x; 1.9612x over previous
import functools

import jax
import jax.numpy as jnp
from jax import lax
from jax.experimental import pallas as pl
from jax.experimental.pallas import tpu as pltpu
from jax.experimental.pallas import tpu_sc as plsc

K = 3
B = 8192
D = 1024
F = 2048
C = 4
OUTW = 128

NC = 2
NS = 16
NW = NC * NS
L = 16

CB = B // NW
T = 256
G = B // T + K
BP = G * T
GEXP = 48
JSUB = 64
NSUB = CB // JSUB

@functools.cache
def _mesh():
    return plsc.VectorSubcoreMesh(
        core_axis_name="c", subcore_axis_name="s",
        num_cores=NC, num_subcores=NS)


def _route_body(e_hbm, j_hbm, pos_hbm, bexp_hbm, jsort_hbm,
                e_all, posbuf, idx2d, jbuf, bexpbuf, sem):
    wid = lax.axis_index("s") * NC + lax.axis_index("c")
    lanes = lax.iota(jnp.int32, L)

    pltpu.sync_copy(e_hbm, e_all)

    myg = wid * (CB // L)

    def count_body(g, carry):
        pre, tot = carry
        ev = e_all[pl.ds(g * L, L)]
        cnt = jnp.zeros((L,), jnp.int32)
        for k in range(K):
            pc = jnp.sum((ev == k).astype(jnp.int32))
            cnt = cnt + jnp.where(lanes == k, lax.broadcast(pc, (L,)),
                                  jnp.zeros((L,), jnp.int32))
        inpre = lax.broadcast((g < myg).astype(jnp.int32), (L,))
        return pre + cnt * inpre, tot + cnt

    zero = jnp.zeros((L,), jnp.int32)
    pre_v, tot_v = lax.fori_loop(0, B // L, count_body, (zero, zero))

    pt = ((tot_v + (T - 1)) // T) * T
    base_v = plsc.cumsum(pt) - pt
    start_v = base_v + pre_v

    def lane_scalar(vec, k):
        return jnp.sum(jnp.where(lanes == k, vec, jnp.zeros((L,), jnp.int32)))

    starts = [lane_scalar(start_v, k) for k in range(K)]

    @pl.when(wid == 0)
    def _():
        bstart = [lane_scalar(base_v, k) // T for k in range(K)]
        nblk = [lane_scalar(pt, k) // T for k in range(K)]
        for gi in range(GEXP // L):
            bb = gi * L + lanes
            v = jnp.zeros((L,), jnp.int32)
            for k in range(1, K):
                lo = lax.broadcast(bstart[k], (L,))
                hi = lax.broadcast(bstart[k] + nblk[k], (L,))
                inb = (bb >= lo) & (bb < hi)
                v = v + jnp.where(inb, jnp.full((L,), k, jnp.int32),
                                  jnp.zeros((L,), jnp.int32))
            bexpbuf[pl.ds(gi * L, L)] = v
        pltpu.sync_copy(bexpbuf, bexp_hbm)

    for g in range(CB // L):
        ev = e_all[pl.ds(wid * CB + g * L, L)]
        posv = jnp.zeros((L,), jnp.int32)
        for k in range(K):
            m = ev == k
            rank = plsc.cumsum(m.astype(jnp.int32))
            sk = lax.broadcast(starts[k], (L,))
            posv = posv + jnp.where(m, sk + rank - 1,
                                    jnp.zeros((L,), jnp.int32))
            starts[k] = starts[k] + jnp.sum(m.astype(jnp.int32))
        posbuf[pl.ds(g * L, L)] = posv
        idx2d[g // (JSUB // L), pl.ds((g % (JSUB // L)) * L, L)] = posv
    pltpu.sync_copy(posbuf, pos_hbm.at[pl.ds(wid * CB, CB)])

    for sub in range(NSUB):
        pltpu.sync_copy(j_hbm.at[pl.ds(wid * CB + sub * JSUB, JSUB)], jbuf)
        pltpu.async_copy(jbuf, jsort_hbm.at[idx2d.at[sub]], sem).wait()


@functools.cache
def _route():
    return pl.kernel(
        _route_body,
        out_type=(jax.ShapeDtypeStruct((B,), jnp.int32),
                  jax.ShapeDtypeStruct((GEXP,), jnp.int32),
                  jax.ShapeDtypeStruct((BP, D), jnp.float32)),
        mesh=_mesh(),
        compiler_params=pltpu.CompilerParams(needs_layout_passes=False),
        scratch_types=[
            pltpu.VMEM((B,), jnp.int32),
            pltpu.VMEM((CB,), jnp.int32),
            pltpu.VMEM((NSUB, JSUB), jnp.int32),
            pltpu.VMEM((JSUB, D), jnp.float32),
            pltpu.VMEM((GEXP,), jnp.int32),
            pltpu.SemaphoreType.DMA,
        ],
    )


def _unperm_body(cq_hbm, pos_hbm, out_hbm, idxbuf, rowbuf, sem):
    wid = lax.axis_index("s") * NC + lax.axis_index("c")
    base = wid * CB
    pltpu.sync_copy(pos_hbm.at[pl.ds(base, CB)], idxbuf)
    pltpu.async_copy(cq_hbm.at[idxbuf], rowbuf, sem).wait()
    pltpu.sync_copy(rowbuf, out_hbm.at[pl.ds(base, CB)])


@functools.cache
def _unperm():
    return pl.kernel(
        _unperm_body,
        out_type=jax.ShapeDtypeStruct((B, OUTW), jnp.float32),
        mesh=_mesh(),
        compiler_params=pltpu.CompilerParams(needs_layout_passes=False),
        scratch_types=[
            pltpu.VMEM((CB,), jnp.int32),
            pltpu.VMEM((CB, OUTW), jnp.float32),
            pltpu.SemaphoreType.DMA,
        ],
    )


def _mlp_body(bexp_ref, j_ref, w1_ref, b1_ref, wcq_ref, bcq_ref, o_ref):
    h = jnp.dot(j_ref[...], w1_ref[0], preferred_element_type=jnp.float32)
    h = jnp.maximum(h + b1_ref[0], 0.0)
    o = jnp.dot(h, wcq_ref[0], preferred_element_type=jnp.float32) + bcq_ref[0]
    col = lax.broadcasted_iota(jnp.int32, (T, OUTW), 1)
    cm = col < C
    qm = (col >= C) & (col < C + 3)
    cmean = jnp.sum(jnp.where(cm, o, 0.0), axis=1, keepdims=True) * (1.0 / C)
    qmean = jnp.sum(jnp.where(qm, o, 0.0), axis=1, keepdims=True) * (1.0 / 3.0)
    o_ref[...] = o - jnp.where(cm, cmean, 0.0) - jnp.where(qm, qmean, 0.0)


def _mlp(bexp, jsort, W1, b1r, Wcq, bcqr):
    grid_spec = pltpu.PrefetchScalarGridSpec(
        num_scalar_prefetch=1,
        grid=(G,),
        in_specs=[
            pl.BlockSpec((T, D), lambda i, be: (i, 0)),
            pl.BlockSpec((1, D, F), lambda i, be: (be[i], 0, 0)),
            pl.BlockSpec((1, 1, F), lambda i, be: (be[i], 0, 0)),
            pl.BlockSpec((1, F, OUTW), lambda i, be: (be[i], 0, 0)),
            pl.BlockSpec((1, 1, OUTW), lambda i, be: (be[i], 0, 0)),
        ],
        out_specs=pl.BlockSpec((T, OUTW), lambda i, be: (i, 0)),
    )
    return pl.pallas_call(
        _mlp_body,
        grid_spec=grid_spec,
        out_shape=jax.ShapeDtypeStruct((BP, OUTW), jnp.float32),
    )(bexp, jsort, W1, b1r, Wcq, bcqr)


def kernel(j, e, W1, b1, Wc, bc, Wq, bq):
    e32 = e.astype(jnp.int32)
    pad = OUTW - C - 3
    Wcq = jnp.concatenate(
        [Wc, Wq, jnp.zeros((K, F, pad), Wc.dtype)], axis=-1)
    bcq = jnp.concatenate(
        [bc, bq, jnp.zeros((K, pad), bc.dtype)], axis=-1).reshape(K, 1, OUTW)
    b1r = b1.reshape(K, 1, F)

    pos, bexp, jsort = _route()(e32, j)
    cq = _mlp(bexp, jsort, W1, b1r, Wcq, bcq)
    out = _unperm()(cq, pos)
    return out[:, :C], out[:, C:C + 3]
